# baseline (speedup 1.0000x reference)
import functools

import jax
import jax.numpy as jnp
from jax import lax
from jax.experimental import pallas as pl
from jax.experimental.pallas import tpu as pltpu
from jax.experimental.pallas import tpu_sc as plsc

B = 16384
D = 32
NC = 2
NS = 16
NW = NC * NS
BPW = B // NW
IDX_MINOR = 128
IDX_ROWS = BPW // IDX_MINOR
MAX_INFLIGHT = 64


def _body(uid_hbm, mid_hbm, ut_hbm, mt_hbm, fc_hbm, out_hbm,
          uidx_v, midx_v, u_v, m_v, out_v, fc_v, sem):
    wid = lax.axis_index("s") * NC + lax.axis_index("c")
    base = wid * IDX_ROWS

    pltpu.sync_copy(uid_hbm.at[pl.ds(base, IDX_ROWS)], uidx_v)
    pltpu.sync_copy(mid_hbm.at[pl.ds(base, IDX_ROWS)], midx_v)
    pltpu.sync_copy(fc_hbm, fc_v)

    pending = []
    for d in range(D):
        for j in range(IDX_ROWS):
            pending.append(pltpu.async_copy(
                ut_hbm.at[d].at[uidx_v.at[j]],
                u_v.at[d, pl.ds(j * IDX_MINOR, IDX_MINOR)], sem))
            pending.append(pltpu.async_copy(
                mt_hbm.at[d].at[midx_v.at[j]],
                m_v.at[d, pl.ds(j * IDX_MINOR, IDX_MINOR)], sem))
            while len(pending) > MAX_INFLIGHT:
                pending.pop(0).wait()
    for cp in pending:
        cp.wait()

    w_vec = fc_v[pl.ds(0, 16)]
    b_vec = fc_v[pl.ds(16, 16)]

    def group(g, carry):
        acc = jnp.zeros((16,), jnp.float32)
        for d in range(D):
            acc = acc + (u_v[d, pl.ds(g * 16, 16)] *
                         m_v[d, pl.ds(g * 16, 16)])
        y = acc * w_vec + b_vec
        out_v[pl.ds(g * 16, 16)] = 1.0 / (1.0 + jnp.exp(-y))
        return carry

    lax.fori_loop(0, BPW // 16, group, 0)

    pltpu.sync_copy(out_v, out_hbm.at[pl.ds(wid * BPW, BPW)])


@jax.jit
def _run(uid2, mid2, ut_t, mt_t, fc128):
    mesh = plsc.VectorSubcoreMesh(core_axis_name="c", subcore_axis_name="s")
    f = functools.partial(
        pl.kernel,
        mesh=mesh,
        compiler_params=pltpu.CompilerParams(needs_layout_passes=False,
                                             use_tc_tiling_on_sc=False),
        out_type=jax.ShapeDtypeStruct((B,), jnp.float32),
        scratch_types=[
            pltpu.VMEM((IDX_ROWS, IDX_MINOR), jnp.int32),
            pltpu.VMEM((IDX_ROWS, IDX_MINOR), jnp.int32),
            pltpu.VMEM((D, BPW), jnp.float32),
            pltpu.VMEM((D, BPW), jnp.float32),
            pltpu.VMEM((BPW,), jnp.float32),
            pltpu.VMEM((128,), jnp.float32),
            pltpu.SemaphoreType.DMA,
        ],
    )(_body)
    return f(uid2, mid2, ut_t, mt_t, fc128)


def kernel(user_ids, movie_ids, u_table, m_table, fc_w, fc_b):
    uid2 = user_ids.astype(jnp.int32).reshape(B // IDX_MINOR, IDX_MINOR)
    mid2 = movie_ids.astype(jnp.int32).reshape(B // IDX_MINOR, IDX_MINOR)
    ut_t = u_table.T
    mt_t = m_table.T
    fc128 = jnp.concatenate([
        jnp.full((16,), fc_w.reshape(()), jnp.float32),
        jnp.full((16,), fc_b.reshape(()), jnp.float32),
        jnp.zeros((96,), jnp.float32),
    ])
    out = _run(uid2, mid2, ut_t, mt_t, fc128)
    return out.reshape(B, 1)


# speedup vs baseline: 10.9627x; 10.9627x over previous
import functools

import jax
import jax.numpy as jnp
from jax import lax
from jax.experimental import pallas as pl
from jax.experimental.pallas import tpu as pltpu
from jax.experimental.pallas import tpu_sc as plsc

B = 16384
D = 32
NC = 2
NS = 16
NW = NC * NS
BPW = B // NW
IDX_MINOR = 128
IDX_ROWS = BPW // IDX_MINOR
MAX_INFLIGHT = 64

UN = 1000000
UNPAD = 1000064
UCQ = 13
MN = 100000
MNPAD = 100096
MCQ = 17

_MESH = dict(core_axis_name="c", subcore_axis_name="s")


def _wid():
    return lax.axis_index("s") * NC + lax.axis_index("c")


def _make_relayout(n, npad, cq):
    qtot = npad // 128
    nch = qtot // cq
    w = cq * 128
    ntasks = -(-nch // 8)

    def body(y_hbm, out_hbm, buf, sem_r, sem_w):
        wid = _wid()
        h = wid // 8
        qw = wid % 8

        def read_cp(t, c):
            return pltpu.make_async_copy(
                y_hbm.at[pl.ds(h * 8, 8), pl.ds(c * w, w)],
                buf.at[t % 2], sem_r)

        def write_cp(t, c, i):
            row0 = (h * 8 + i) * npad + c * w
            return pltpu.make_async_copy(
                buf.at[t % 2, i],
                out_hbm.at[pl.ds(row0, w)], sem_w)

        def task(t, carry):
            c2 = qw + 8 * (t - 2)

            @pl.when(jnp.logical_and(t >= 2, c2 < nch))
            def _():
                for i in range(8):
                    write_cp(t - 2, c2, i).wait()

            c = qw + 8 * t

            @pl.when(c < nch)
            def _():
                read_cp(t, c).start()
                read_cp(t, c).wait()
                for i in range(8):
                    write_cp(t, c, i).start()
            return carry

        lax.fori_loop(0, ntasks + 2, task, 0)

    mesh = plsc.VectorSubcoreMesh(**_MESH)
    return functools.partial(
        pl.kernel,
        mesh=mesh,
        compiler_params=pltpu.CompilerParams(needs_layout_passes=False,
                                             use_tc_tiling_on_sc=True),
        out_type=jax.ShapeDtypeStruct((32 * npad,), jnp.float32),
        scratch_types=[
            pltpu.VMEM((2, 8, w), jnp.float32),
            pltpu.SemaphoreType.DMA,
            pltpu.SemaphoreType.DMA,
        ],
    )(body)


def _gather_body(uid_hbm, mid_hbm, ut_hbm, mt_hbm, fc_hbm, out_hbm,
                 uidx_v, midx_v, uflat_v, mflat_v, u_v, m_v, out_v, fc_v,
                 sem):
    wid = _wid()
    base = wid * IDX_ROWS

    pltpu.sync_copy(uid_hbm.at[pl.ds(base, IDX_ROWS)], uidx_v)
    pltpu.sync_copy(mid_hbm.at[pl.ds(base, IDX_ROWS)], midx_v)
    pltpu.sync_copy(fc_hbm, fc_v)

    for j in range(IDX_ROWS):
        for k in range(IDX_MINOR // 16):
            ublk = uidx_v[j, pl.ds(k * 16, 16)]
            mblk = midx_v[j, pl.ds(k * 16, 16)]
            for d in range(D):
                uflat_v[d, j, pl.ds(k * 16, 16)] = ublk + d * UNPAD
                mflat_v[d, j, pl.ds(k * 16, 16)] = mblk + d * MNPAD

    pending = []
    for d in range(D):
        for j in range(IDX_ROWS):
            pending.append(pltpu.async_copy(
                ut_hbm.at[uflat_v.at[d, j]],
                u_v.at[d, pl.ds(j * IDX_MINOR, IDX_MINOR)], sem))
            pending.append(pltpu.async_copy(
                mt_hbm.at[mflat_v.at[d, j]],
                m_v.at[d, pl.ds(j * IDX_MINOR, IDX_MINOR)], sem))
            while len(pending) > MAX_INFLIGHT:
                pending.pop(0).wait()
    for cp in pending:
        cp.wait()

    w_vec = fc_v[pl.ds(0, 16)]
    b_vec = fc_v[pl.ds(16, 16)]

    def group(g, carry):
        acc = jnp.zeros((16,), jnp.float32)
        for d in range(D):
            acc = acc + (u_v[d, pl.ds(g * 16, 16)] *
                         m_v[d, pl.ds(g * 16, 16)])
        y = acc * w_vec + b_vec
        out_v[pl.ds(g * 16, 16)] = 1.0 / (1.0 + jnp.exp(-y))
        return carry

    lax.fori_loop(0, BPW // 16, group, 0)

    pltpu.sync_copy(out_v, out_hbm.at[pl.ds(wid * BPW, BPW)])


def _make_gather():
    mesh = plsc.VectorSubcoreMesh(**_MESH)
    return functools.partial(
        pl.kernel,
        mesh=mesh,
        compiler_params=pltpu.CompilerParams(needs_layout_passes=False,
                                             use_tc_tiling_on_sc=False),
        out_type=jax.ShapeDtypeStruct((B,), jnp.float32),
        scratch_types=[
            pltpu.VMEM((IDX_ROWS, IDX_MINOR), jnp.int32),
            pltpu.VMEM((IDX_ROWS, IDX_MINOR), jnp.int32),
            pltpu.VMEM((D, IDX_ROWS, IDX_MINOR), jnp.int32),
            pltpu.VMEM((D, IDX_ROWS, IDX_MINOR), jnp.int32),
            pltpu.VMEM((D, BPW), jnp.float32),
            pltpu.VMEM((D, BPW), jnp.float32),
            pltpu.VMEM((BPW,), jnp.float32),
            pltpu.VMEM((128,), jnp.float32),
            pltpu.SemaphoreType.DMA,
        ],
    )(_gather_body)


@jax.jit
def _run(uid2, mid2, ut_t, mt_t, fc128):
    u_lin = _make_relayout(UN, UNPAD, UCQ)(ut_t)
    m_lin = _make_relayout(MN, MNPAD, MCQ)(mt_t)
    return _make_gather()(uid2, mid2, u_lin, m_lin, fc128)


def kernel(user_ids, movie_ids, u_table, m_table, fc_w, fc_b):
    uid2 = user_ids.astype(jnp.int32).reshape(B // IDX_MINOR, IDX_MINOR)
    mid2 = movie_ids.astype(jnp.int32).reshape(B // IDX_MINOR, IDX_MINOR)
    ut_t = u_table.T
    mt_t = m_table.T
    fc128 = jnp.concatenate([
        jnp.full((16,), fc_w.reshape(()), jnp.float32),
        jnp.full((16,), fc_b.reshape(()), jnp.float32),
        jnp.zeros((96,), jnp.float32),
    ])
    out = _run(uid2, mid2, ut_t, mt_t, fc128)
    return out.reshape(B, 1)


# speedup vs baseline: 11.3969x; 1.0396x over previous
import functools

import jax
import jax.numpy as jnp
from jax import lax
from jax.experimental import pallas as pl
from jax.experimental.pallas import tpu as pltpu
from jax.experimental.pallas import tpu_sc as plsc

B = 16384
D = 32
NC = 2
NS = 16
NW = NC * NS
BPW = B // NW
IDX_MINOR = 128
IDX_ROWS = BPW // IDX_MINOR

UN = 1000000
UNPAD = 1000064
UCQ = 13
MN = 100000
MNPAD = 100096
MCQ = 17

_MESH = dict(core_axis_name="c", subcore_axis_name="s")


def _wid():
    return lax.axis_index("s") * NC + lax.axis_index("c")


def _relayout_table(y_hbm, out_hbm, buf, sem_r, sem_w, npad, cq, h, qw):
    qtot = npad // 128
    nch = qtot // cq
    w = cq * 128
    ntasks = -(-nch // 8)

    def read_cp(t, c):
        return pltpu.make_async_copy(
            y_hbm.at[pl.ds(h * 8, 8), pl.ds(c * w, w)],
            buf.at[t % 2], sem_r)

    def write_cp(t, c, i):
        row0 = (h * 8 + i) * npad + c * w
        return pltpu.make_async_copy(
            buf.at[t % 2, i],
            out_hbm.at[pl.ds(row0, w)], sem_w)

    def task(t, carry):
        c2 = qw + 8 * (t - 2)

        @pl.when(jnp.logical_and(t >= 2, c2 < nch))
        def _():
            for i in range(8):
                write_cp(t - 2, c2, i).wait()

        c = qw + 8 * t

        @pl.when(c < nch)
        def _():
            read_cp(t, c).start()
            read_cp(t, c).wait()
            for i in range(8):
                write_cp(t, c, i).start()
        return carry

    lax.fori_loop(0, ntasks + 2, task, 0)


def _relayout_body(yu_hbm, ym_hbm, lu_hbm, lm_hbm, ubuf, mbuf,
                   sem_r, sem_w):
    wid = _wid()
    h = wid // 8
    qw = wid % 8
    _relayout_table(yu_hbm, lu_hbm, ubuf, sem_r, sem_w, UNPAD, UCQ, h, qw)
    _relayout_table(ym_hbm, lm_hbm, mbuf, sem_r, sem_w, MNPAD, MCQ, h, qw)


def _make_relayout():
    mesh = plsc.VectorSubcoreMesh(**_MESH)
    return functools.partial(
        pl.kernel,
        mesh=mesh,
        compiler_params=pltpu.CompilerParams(needs_layout_passes=False,
                                             use_tc_tiling_on_sc=True),
        out_type=(jax.ShapeDtypeStruct((32 * UNPAD,), jnp.float32),
                  jax.ShapeDtypeStruct((32 * MNPAD,), jnp.float32)),
        scratch_types=[
            pltpu.VMEM((2, 8, UCQ * 128), jnp.float32),
            pltpu.VMEM((2, 8, MCQ * 128), jnp.float32),
            pltpu.SemaphoreType.DMA,
            pltpu.SemaphoreType.DMA,
        ],
    )(_relayout_body)


def _gather_body(uid_hbm, mid_hbm, ut_hbm, mt_hbm, fc_hbm, out_hbm,
                 uidx_v, midx_v, uflat_v, mflat_v, u_v, m_v, out_v, fc_v,
                 sem):
    wid = _wid()
    base = wid * IDX_ROWS

    pltpu.sync_copy(uid_hbm.at[pl.ds(base, IDX_ROWS)], uidx_v)
    pltpu.sync_copy(mid_hbm.at[pl.ds(base, IDX_ROWS)], midx_v)
    pltpu.sync_copy(fc_hbm, fc_v)

    for j in range(IDX_ROWS):
        for k in range(IDX_MINOR // 16):
            off = j * IDX_MINOR + k * 16
            ublk = uidx_v[j, pl.ds(k * 16, 16)]
            mblk = midx_v[j, pl.ds(k * 16, 16)]
            for d in range(D):
                uflat_v[d, pl.ds(off, 16)] = ublk + d * UNPAD
                mflat_v[d, pl.ds(off, 16)] = mblk + d * MNPAD

    pending = []
    for d in range(D):
        pending.append(pltpu.async_copy(
            ut_hbm.at[uflat_v.at[d]], u_v.at[d], sem))
        pending.append(pltpu.async_copy(
            mt_hbm.at[mflat_v.at[d]], m_v.at[d], sem))
    for cp in pending:
        cp.wait()

    w_vec = fc_v[pl.ds(0, 16)]
    b_vec = fc_v[pl.ds(16, 16)]

    def group(g, carry):
        acc = jnp.zeros((16,), jnp.float32)
        for d in range(D):
            acc = acc + (u_v[d, pl.ds(g * 16, 16)] *
                         m_v[d, pl.ds(g * 16, 16)])
        y = acc * w_vec + b_vec
        out_v[pl.ds(g * 16, 16)] = 1.0 / (1.0 + jnp.exp(-y))
        return carry

    lax.fori_loop(0, BPW // 16, group, 0)

    pltpu.sync_copy(out_v, out_hbm.at[pl.ds(wid * BPW, BPW)])


def _make_gather():
    mesh = plsc.VectorSubcoreMesh(**_MESH)
    return functools.partial(
        pl.kernel,
        mesh=mesh,
        compiler_params=pltpu.CompilerParams(needs_layout_passes=False,
                                             use_tc_tiling_on_sc=False),
        out_type=jax.ShapeDtypeStruct((B,), jnp.float32),
        scratch_types=[
            pltpu.VMEM((IDX_ROWS, IDX_MINOR), jnp.int32),
            pltpu.VMEM((IDX_ROWS, IDX_MINOR), jnp.int32),
            pltpu.VMEM((D, BPW), jnp.int32),
            pltpu.VMEM((D, BPW), jnp.int32),
            pltpu.VMEM((D, BPW), jnp.float32),
            pltpu.VMEM((D, BPW), jnp.float32),
            pltpu.VMEM((BPW,), jnp.float32),
            pltpu.VMEM((128,), jnp.float32),
            pltpu.SemaphoreType.DMA,
        ],
    )(_gather_body)


@jax.jit
def _run(uid2, mid2, ut_t, mt_t, fc128):
    u_lin, m_lin = _make_relayout()(ut_t, mt_t)
    return _make_gather()(uid2, mid2, u_lin, m_lin, fc128)


def kernel(user_ids, movie_ids, u_table, m_table, fc_w, fc_b):
    uid2 = user_ids.astype(jnp.int32).reshape(B // IDX_MINOR, IDX_MINOR)
    mid2 = movie_ids.astype(jnp.int32).reshape(B // IDX_MINOR, IDX_MINOR)
    ut_t = u_table.T
    mt_t = m_table.T
    fc128 = jnp.concatenate([
        jnp.full((16,), fc_w.reshape(()), jnp.float32),
        jnp.full((16,), fc_b.reshape(()), jnp.float32),
        jnp.zeros((96,), jnp.float32),
    ])
    out = _run(uid2, mid2, ut_t, mt_t, fc128)
    return out.reshape(B, 1)


# speedup vs baseline: 15.3620x; 1.3479x over previous
import functools

import jax
import jax.numpy as jnp
from jax import lax
from jax.experimental import pallas as pl
from jax.experimental.pallas import tpu as pltpu
from jax.experimental.pallas import tpu_sc as plsc

B = 16384
D = 32
NC = 2
NS = 16
NW = NC * NS
BPW = B // NW
IDX_MINOR = 128
IDX_ROWS = BPW // IDX_MINOR

UN = 1000000
UNPAD = 1000064
UCQ = 13
MN = 100000
MNPAD = 100096
MCQ = 17

_MESH = dict(core_axis_name="c", subcore_axis_name="s")


def _wid():
    return lax.axis_index("s") * NC + lax.axis_index("c")


def _relayout_table(y_hbm, out_hbm, buf, sem_r, sem_w, npad, cq, h, qw):
    qtot = npad // 128
    nch = qtot // cq
    w = cq * 128
    ntasks = -(-nch // 8)

    def read_cp(t, c):
        return pltpu.make_async_copy(
            y_hbm.at[pl.ds(h * 8, 8), pl.ds(c * w, w)],
            buf.at[t % 3], sem_r)

    def write_cp(t, c, i):
        row0 = (h * 8 + i) * npad + c * w
        return pltpu.make_async_copy(
            buf.at[t % 3, i],
            out_hbm.at[pl.ds(row0, w)], sem_w)

    read_cp(0, qw).start()

    def task(t, carry):
        c2 = qw + 8 * (t - 2)

        @pl.when(jnp.logical_and(t >= 2, c2 < nch))
        def _():
            for i in range(8):
                write_cp(t - 2, c2, i).wait()

        c = qw + 8 * t
        cn = c + 8

        @pl.when(cn < nch)
        def _():
            read_cp(t + 1, cn).start()

        @pl.when(c < nch)
        def _():
            read_cp(t, c).wait()
            for i in range(8):
                write_cp(t, c, i).start()
        return carry

    lax.fori_loop(0, ntasks + 2, task, 0)


def _relayout_body(yu_hbm, ym_hbm, lu_hbm, lm_hbm, ubuf, mbuf,
                   sem_r, sem_w):
    wid = _wid()
    h = wid // 8
    qw = wid % 8
    _relayout_table(yu_hbm, lu_hbm, ubuf, sem_r, sem_w, UNPAD, UCQ, h, qw)
    _relayout_table(ym_hbm, lm_hbm, mbuf, sem_r, sem_w, MNPAD, MCQ, h, qw)


def _make_relayout():
    mesh = plsc.VectorSubcoreMesh(**_MESH)
    return functools.partial(
        pl.kernel,
        mesh=mesh,
        compiler_params=pltpu.CompilerParams(needs_layout_passes=False,
                                             use_tc_tiling_on_sc=True),
        out_type=(jax.ShapeDtypeStruct((32 * UNPAD,), jnp.float32),
                  jax.ShapeDtypeStruct((32 * MNPAD,), jnp.float32)),
        scratch_types=[
            pltpu.VMEM((3, 8, UCQ * 128), jnp.float32),
            pltpu.VMEM((3, 8, MCQ * 128), jnp.float32),
            pltpu.SemaphoreType.DMA,
            pltpu.SemaphoreType.DMA,
        ],
    )(_relayout_body)


def _gather_body(uid_hbm, mid_hbm, ut_hbm, mt_hbm, fc_hbm, out_hbm,
                 uidx_v, midx_v, uflat_v, mflat_v, u_v, m_v, out_v, fc_v,
                 sem):
    wid = _wid()
    base = wid * IDX_ROWS

    pltpu.sync_copy(uid_hbm.at[pl.ds(base, IDX_ROWS)], uidx_v)
    pltpu.sync_copy(mid_hbm.at[pl.ds(base, IDX_ROWS)], midx_v)
    pltpu.sync_copy(fc_hbm, fc_v)

    for j in range(IDX_ROWS):
        for k in range(IDX_MINOR // 16):
            off = j * IDX_MINOR + k * 16
            ublk = uidx_v[j, pl.ds(k * 16, 16)]
            mblk = midx_v[j, pl.ds(k * 16, 16)]
            for d in range(D):
                uflat_v[pl.ds(d * BPW + off, 16)] = ublk + d * UNPAD
                mflat_v[pl.ds(d * BPW + off, 16)] = mblk + d * MNPAD

    cp_u = pltpu.async_copy(ut_hbm.at[uflat_v], u_v, sem)
    cp_m = pltpu.async_copy(mt_hbm.at[mflat_v], m_v, sem)
    cp_u.wait()
    cp_m.wait()

    w_vec = fc_v[pl.ds(0, 16)]
    b_vec = fc_v[pl.ds(16, 16)]

    def group(g, carry):
        acc = jnp.zeros((16,), jnp.float32)
        for d in range(D):
            acc = acc + (u_v[pl.ds(d * BPW + g * 16, 16)] *
                         m_v[pl.ds(d * BPW + g * 16, 16)])
        y = acc * w_vec + b_vec
        out_v[pl.ds(g * 16, 16)] = 1.0 / (1.0 + jnp.exp(-y))
        return carry

    lax.fori_loop(0, BPW // 16, group, 0)

    pltpu.sync_copy(out_v, out_hbm.at[pl.ds(wid * BPW, BPW)])


def _make_gather():
    mesh = plsc.VectorSubcoreMesh(**_MESH)
    return functools.partial(
        pl.kernel,
        mesh=mesh,
        compiler_params=pltpu.CompilerParams(needs_layout_passes=False,
                                             use_tc_tiling_on_sc=False),
        out_type=jax.ShapeDtypeStruct((B,), jnp.float32),
        scratch_types=[
            pltpu.VMEM((IDX_ROWS, IDX_MINOR), jnp.int32),
            pltpu.VMEM((IDX_ROWS, IDX_MINOR), jnp.int32),
            pltpu.VMEM((D * BPW,), jnp.int32),
            pltpu.VMEM((D * BPW,), jnp.int32),
            pltpu.VMEM((D * BPW,), jnp.float32),
            pltpu.VMEM((D * BPW,), jnp.float32),
            pltpu.VMEM((BPW,), jnp.float32),
            pltpu.VMEM((128,), jnp.float32),
            pltpu.SemaphoreType.DMA,
        ],
    )(_gather_body)


@jax.jit
def _run(uid2, mid2, ut_t, mt_t, fc128):
    u_lin, m_lin = _make_relayout()(ut_t, mt_t)
    return _make_gather()(uid2, mid2, u_lin, m_lin, fc128)


def kernel(user_ids, movie_ids, u_table, m_table, fc_w, fc_b):
    uid2 = user_ids.astype(jnp.int32).reshape(B // IDX_MINOR, IDX_MINOR)
    mid2 = movie_ids.astype(jnp.int32).reshape(B // IDX_MINOR, IDX_MINOR)
    ut_t = u_table.T
    mt_t = m_table.T
    fc128 = jnp.concatenate([
        jnp.full((16,), fc_w.reshape(()), jnp.float32),
        jnp.full((16,), fc_b.reshape(()), jnp.float32),
        jnp.zeros((96,), jnp.float32),
    ])
    out = _run(uid2, mid2, ut_t, mt_t, fc128)
    return out.reshape(B, 1)


# speedup vs baseline: 15.6455x; 1.0185x over previous
import functools

import jax
import jax.numpy as jnp
from jax import lax
from jax.experimental import pallas as pl
from jax.experimental.pallas import tpu as pltpu
from jax.experimental.pallas import tpu_sc as plsc

B = 16384
D = 32
NC = 2
NS = 16
NW = NC * NS
BPW = B // NW
IDX_MINOR = 128
IDX_ROWS = BPW // IDX_MINOR

UN = 1000000
UNPAD = 1000064
UCQ = 13
MN = 100000
MNPAD = 100096
MCQ = 17

_MESH = dict(core_axis_name="c", subcore_axis_name="s")


def _wid():
    return lax.axis_index("s") * NC + lax.axis_index("c")


def _relayout_table(y_hbm, out_hbm, buf, sem_r, sem_w, npad, cq, h, qw):
    qtot = npad // 128
    nch = qtot // cq
    w = cq * 128
    ntasks = -(-nch // 8)

    def read_cp(t, c):
        return pltpu.make_async_copy(
            y_hbm.at[pl.ds(h * 8, 8), pl.ds(c * w, w)],
            buf.at[t % 4], sem_r)

    def write_cp(t, c, i):
        row0 = (h * 8 + i) * npad + c * w
        return pltpu.make_async_copy(
            buf.at[t % 4, i],
            out_hbm.at[pl.ds(row0, w)], sem_w)

    read_cp(0, qw).start()
    read_cp(1, qw + 8).start()

    def task(t, carry):
        c2 = qw + 8 * (t - 2)

        @pl.when(jnp.logical_and(t >= 2, c2 < nch))
        def _():
            for i in range(8):
                write_cp(t - 2, c2, i).wait()

        c = qw + 8 * t
        cn = c + 16

        @pl.when(cn < nch)
        def _():
            read_cp(t + 2, cn).start()

        @pl.when(c < nch)
        def _():
            read_cp(t, c).wait()
            for i in range(8):
                write_cp(t, c, i).start()
        return carry

    lax.fori_loop(0, ntasks + 2, task, 0)


def _relayout_body(yu_hbm, ym_hbm, lu_hbm, lm_hbm, ubuf, mbuf,
                   sem_r, sem_w):
    wid = _wid()
    h = wid // 8
    qw = wid % 8
    _relayout_table(yu_hbm, lu_hbm, ubuf, sem_r, sem_w, UNPAD, UCQ, h, qw)
    _relayout_table(ym_hbm, lm_hbm, mbuf, sem_r, sem_w, MNPAD, MCQ, h, qw)


def _make_relayout():
    mesh = plsc.VectorSubcoreMesh(**_MESH)
    return functools.partial(
        pl.kernel,
        mesh=mesh,
        compiler_params=pltpu.CompilerParams(needs_layout_passes=False,
                                             use_tc_tiling_on_sc=True),
        out_type=(jax.ShapeDtypeStruct((32 * UNPAD,), jnp.float32),
                  jax.ShapeDtypeStruct((32 * MNPAD,), jnp.float32)),
        scratch_types=[
            pltpu.VMEM((3, 8, UCQ * 128), jnp.float32),
            pltpu.VMEM((3, 8, MCQ * 128), jnp.float32),
            pltpu.SemaphoreType.DMA,
            pltpu.SemaphoreType.DMA,
        ],
    )(_relayout_body)


def _gather_body(uid_hbm, mid_hbm, ut_hbm, mt_hbm, fc_hbm, out_hbm,
                 uidx_v, midx_v, uflat_v, mflat_v, u_v, m_v, out_v, fc_v,
                 sem):
    wid = _wid()
    base = wid * IDX_ROWS

    pltpu.sync_copy(uid_hbm.at[pl.ds(base, IDX_ROWS)], uidx_v)
    pltpu.sync_copy(mid_hbm.at[pl.ds(base, IDX_ROWS)], midx_v)
    pltpu.sync_copy(fc_hbm, fc_v)

    for j in range(IDX_ROWS):
        for k in range(IDX_MINOR // 16):
            off = j * IDX_MINOR + k * 16
            ublk = uidx_v[j, pl.ds(k * 16, 16)]
            mblk = midx_v[j, pl.ds(k * 16, 16)]
            for d in range(D):
                uflat_v[pl.ds(d * BPW + off, 16)] = ublk + d * UNPAD
                mflat_v[pl.ds(d * BPW + off, 16)] = mblk + d * MNPAD

    cp_u = pltpu.async_copy(ut_hbm.at[uflat_v], u_v, sem)
    cp_m = pltpu.async_copy(mt_hbm.at[mflat_v], m_v, sem)
    cp_u.wait()
    cp_m.wait()

    w_vec = fc_v[pl.ds(0, 16)]
    b_vec = fc_v[pl.ds(16, 16)]

    def group(g, carry):
        acc = jnp.zeros((16,), jnp.float32)
        for d in range(D):
            acc = acc + (u_v[pl.ds(d * BPW + g * 16, 16)] *
                         m_v[pl.ds(d * BPW + g * 16, 16)])
        y = acc * w_vec + b_vec
        out_v[pl.ds(g * 16, 16)] = 1.0 / (1.0 + jnp.exp(-y))
        return carry

    lax.fori_loop(0, BPW // 16, group, 0)

    pltpu.sync_copy(out_v, out_hbm.at[pl.ds(wid * BPW, BPW)])


def _make_gather():
    mesh = plsc.VectorSubcoreMesh(**_MESH)
    return functools.partial(
        pl.kernel,
        mesh=mesh,
        compiler_params=pltpu.CompilerParams(needs_layout_passes=False,
                                             use_tc_tiling_on_sc=False),
        out_type=jax.ShapeDtypeStruct((B,), jnp.float32),
        scratch_types=[
            pltpu.VMEM((IDX_ROWS, IDX_MINOR), jnp.int32),
            pltpu.VMEM((IDX_ROWS, IDX_MINOR), jnp.int32),
            pltpu.VMEM((D * BPW,), jnp.int32),
            pltpu.VMEM((D * BPW,), jnp.int32),
            pltpu.VMEM((D * BPW,), jnp.float32),
            pltpu.VMEM((D * BPW,), jnp.float32),
            pltpu.VMEM((BPW,), jnp.float32),
            pltpu.VMEM((128,), jnp.float32),
            pltpu.SemaphoreType.DMA,
        ],
    )(_gather_body)


@jax.jit
def _run(uid2, mid2, ut_t, mt_t, fc128):
    u_lin, m_lin = _make_relayout()(ut_t, mt_t)
    return _make_gather()(uid2, mid2, u_lin, m_lin, fc128)


def kernel(user_ids, movie_ids, u_table, m_table, fc_w, fc_b):
    uid2 = user_ids.astype(jnp.int32).reshape(B // IDX_MINOR, IDX_MINOR)
    mid2 = movie_ids.astype(jnp.int32).reshape(B // IDX_MINOR, IDX_MINOR)
    ut_t = u_table.T
    mt_t = m_table.T
    fc128 = jnp.concatenate([
        jnp.full((16,), fc_w.reshape(()), jnp.float32),
        jnp.full((16,), fc_b.reshape(()), jnp.float32),
        jnp.zeros((96,), jnp.float32),
    ])
    out = _run(uid2, mid2, ut_t, mt_t, fc128)
    return out.reshape(B, 1)
